# baseline (speedup 1.0000x reference)
import functools

import jax
import jax.numpy as jnp
import numpy as np
from jax import lax
from jax.experimental import pallas as pl
from jax.experimental.pallas import tpu as pltpu
from jax.experimental.pallas import tpu_sc as plsc

_B = 16384
_C = 128
_K = 24
_DELTA_OMEGA = float(np.log(30.0) / 23.0)
_TAU_OFFSET = float(np.log(24.0))

_NC = 2
_NS = 16
_NW = _NC * _NS
_RPT = _B // _NW
_CH = 16
_NCHUNK = _RPT // _CH
_ROWP = _C * _K + 1


def _idx_body(c0_ref, c1_ref, kl_ref, al_ref):
    c0 = c0_ref[...]
    c1 = c1_ref[...]
    geo = jnp.maximum(jnp.sqrt(c0 * c1), 1e-10)
    tau = (-jnp.log(geo) - _TAU_OFFSET) / _DELTA_OMEGA
    klf = jnp.clip(jnp.floor(tau), 0.0, float(_K - 2))
    kl_ref[...] = klf.astype(jnp.int32)
    al_ref[...] = jnp.clip(tau - klf, 0.0, 1.0)


def _compute_indices(c0, c1):
    return pl.pallas_call(
        _idx_body,
        out_shape=(
            jax.ShapeDtypeStruct(c0.shape, jnp.int32),
            jax.ShapeDtypeStruct(c0.shape, jnp.float32),
        ),
    )(c0, c1)


def _sc_body(h2, klh, alh, wrep, brep, out,
             hbuf, klbuf, albuf, wbuf, bbuf, obuf, sem0, sem1):
    wid = lax.axis_index("s") * _NC + lax.axis_index("c")
    base = wid * _RPT

    pltpu.sync_copy(klh.at[pl.ds(base, _RPT)], klbuf)
    pltpu.sync_copy(alh.at[pl.ds(base, _RPT)], albuf)
    pltpu.sync_copy(wrep, wbuf)
    pltpu.sync_copy(brep, bbuf)

    sems = (sem0, sem1)

    def start(ch, buf_i):
        pltpu.async_copy(
            h2.at[pl.ds(base + ch * _CH, _CH), :],
            hbuf.at[buf_i, :, pl.ds(0, _C * _K)],
            sems[buf_i],
        )

    def wait(buf_i):
        pltpu.make_async_copy(
            h2.at[pl.ds(0, _CH), :],
            hbuf.at[buf_i, :, pl.ds(0, _C * _K)],
            sems[buf_i],
        ).wait()

    start(0, 0)
    start(1, 1)

    iota = lax.iota(jnp.int32, _CH)

    def process(ch, buf_i):
        wait(buf_i)
        hb = hbuf.at[buf_i]
        kl16 = klbuf[pl.ds(ch * _CH, _CH)]
        al16 = albuf[pl.ds(ch * _CH, _CH)]
        acc0 = bbuf[0, :]
        acc1 = bbuf[1, :]
        acc2 = bbuf[2, :]
        for c in range(_C):
            col = kl16 + (c * _K)
            glo = plsc.load_gather(hb, [iota, col])
            ghi = plsc.load_gather(hb, [iota, col + 1])
            hr = glo + al16 * (ghi - glo)
            acc0 = acc0 + hr * wbuf[0, c, :]
            acc1 = acc1 + hr * wbuf[1, c, :]
            acc2 = acc2 + hr * wbuf[2, c, :]
        obuf[0, pl.ds(ch * _CH, _CH)] = acc0
        obuf[1, pl.ds(ch * _CH, _CH)] = acc1
        obuf[2, pl.ds(ch * _CH, _CH)] = acc2

        @pl.when(ch + 2 < _NCHUNK)
        def _():
            start(ch + 2, buf_i)

    def chunk_pair(ch2, carry):
        process(2 * ch2, 0)
        process(2 * ch2 + 1, 1)
        return carry

    lax.fori_loop(0, _NCHUNK // 2, chunk_pair, 0)

    pltpu.sync_copy(obuf, out.at[:, pl.ds(base, _RPT)])


@functools.partial(jax.jit, static_argnames=())
def _sc_call(h2, kl, al, wrep, brep):
    mesh = plsc.VectorSubcoreMesh(core_axis_name="c", subcore_axis_name="s")
    return pl.kernel(
        _sc_body,
        out_type=jax.ShapeDtypeStruct((3, _B), jnp.float32),
        mesh=mesh,
        compiler_params=pltpu.CompilerParams(use_tc_tiling_on_sc=False,
                                              needs_layout_passes=False),
        scratch_types=[
            pltpu.VMEM((2, _CH, _ROWP), jnp.float32),
            pltpu.VMEM((_RPT,), jnp.int32),
            pltpu.VMEM((_RPT,), jnp.float32),
            pltpu.VMEM((3, _C, _CH), jnp.float32),
            pltpu.VMEM((3, _CH), jnp.float32),
            pltpu.VMEM((3, _RPT), jnp.float32),
            pltpu.SemaphoreType.DMA,
            pltpu.SemaphoreType.DMA,
        ],
    )(h2, kl, al, wrep, brep)


def kernel(h, cell, W, b):
    Bn, C_h, Kn = h.shape
    h2 = h.reshape(Bn, C_h * Kn)
    c0 = cell[:, 0].reshape(8, -1)
    c1 = cell[:, 1].reshape(8, -1)
    kl2, al2 = _compute_indices(c0, c1)
    kl = kl2.reshape(Bn)
    al = al2.reshape(Bn)
    wrep = jnp.broadcast_to(W[:, :, None], (3, C_h, _CH)).astype(jnp.float32)
    brep = jnp.broadcast_to(b[:, None], (3, _CH)).astype(jnp.float32)
    o3 = _sc_call(h2, kl, al, wrep, brep)
    return o3.T


# speedup vs baseline: 8.7486x; 8.7486x over previous
import functools

import jax
import jax.numpy as jnp
import numpy as np
from jax import lax
from jax.experimental import pallas as pl
from jax.experimental.pallas import tpu as pltpu
from jax.experimental.pallas import tpu_sc as plsc

_B = 16384
_C = 128
_K = 24
_DELTA_OMEGA = float(np.log(30.0) / 23.0)
_TAU_OFFSET = float(np.log(24.0))

_NC = 2
_NS = 16
_NW = _NC * _NS
_RPT = _B // _NW
_CHB = 128
_NCHB = _RPT // _CHB


def _idx_body(c0_ref, c1_ref, kl_ref, al_ref):
    c0 = c0_ref[...]
    c1 = c1_ref[...]
    geo = jnp.maximum(jnp.sqrt(c0 * c1), 1e-10)
    tau = (-jnp.log(geo) - _TAU_OFFSET) / _DELTA_OMEGA
    klf = jnp.clip(jnp.floor(tau), 0.0, float(_K - 2))
    kl_ref[...] = klf.astype(jnp.int32)
    al_ref[...] = jnp.clip(tau - klf, 0.0, 1.0)


def _compute_indices(c0, c1):
    return pl.pallas_call(
        _idx_body,
        out_shape=(
            jax.ShapeDtypeStruct(c0.shape, jnp.int32),
            jax.ShapeDtypeStruct(c0.shape, jnp.float32),
        ),
    )(c0, c1)


def _sc_body(tab, klh, alh, out,
             klbuf, albuf, ilo, ihi, dlo, dhi, obuf, sem0, sem1):
    wid = lax.axis_index("s") * _NC + lax.axis_index("c")
    base = wid * _RPT

    pltpu.sync_copy(klh.at[pl.ds(base, _RPT)], klbuf)
    pltpu.sync_copy(alh.at[pl.ds(base, _RPT)], albuf)

    iota = lax.iota(jnp.int32, 16)

    def chunk(chb, carry):
        rb = base + chb * _CHB

        def build(g, c):
            kl16 = klbuf[pl.ds(chb * _CHB + g * 16, 16)]
            idx = (rb + g * 16 + iota) * _K + kl16
            ilo[pl.ds(g * 16, 16)] = idx
            ihi[pl.ds(g * 16, 16)] = idx + 1
            return c

        lax.fori_loop(0, _CHB // 16, build, 0)

        c_lo = pltpu.async_copy(tab.at[ilo], dlo, sem0)
        c_hi = pltpu.async_copy(tab.at[ihi], dhi, sem1)
        c_lo.wait()
        c_hi.wait()

        def row(i, c):
            a16 = plsc.load_gather(
                albuf, [jnp.broadcast_to(chb * _CHB + i, (16,)).astype(jnp.int32)])
            for g in range(_C // 16):
                lo = dlo[i, pl.ds(g * 16, 16)]
                hi = dhi[i, pl.ds(g * 16, 16)]
                obuf[i, pl.ds(g * 16, 16)] = lo + a16 * (hi - lo)
            return c

        lax.fori_loop(0, _CHB, row, 0)

        pltpu.sync_copy(obuf, out.at[pl.ds(rb, _CHB), :])
        return carry

    lax.fori_loop(0, _NCHB, chunk, 0)


def _sc_call(tab, kl, al):
    mesh = plsc.VectorSubcoreMesh(core_axis_name="c", subcore_axis_name="s")
    return pl.kernel(
        _sc_body,
        out_type=jax.ShapeDtypeStruct((_B, _C), jnp.float32),
        mesh=mesh,
        compiler_params=pltpu.CompilerParams(use_tc_tiling_on_sc=False,
                                             needs_layout_passes=False),
        scratch_types=[
            pltpu.VMEM((_RPT,), jnp.int32),
            pltpu.VMEM((_RPT,), jnp.float32),
            pltpu.VMEM((_CHB,), jnp.int32),
            pltpu.VMEM((_CHB,), jnp.int32),
            pltpu.VMEM((_CHB, _C), jnp.float32),
            pltpu.VMEM((_CHB, _C), jnp.float32),
            pltpu.VMEM((_CHB, _C), jnp.float32),
            pltpu.SemaphoreType.DMA,
            pltpu.SemaphoreType.DMA,
        ],
    )(tab, kl, al)


def _mv_body(x_ref, w_ref, b_ref, o_ref):
    o_ref[...] = (
        jnp.dot(x_ref[...], w_ref[...], preferred_element_type=jnp.float32)
        + b_ref[...]
    )


def _matvec(x, w8, b8):
    nb = 16
    bm = _B // nb
    return pl.pallas_call(
        _mv_body,
        grid=(nb,),
        in_specs=[
            pl.BlockSpec((bm, _C), lambda i: (i, 0)),
            pl.BlockSpec((_C, 8), lambda i: (0, 0)),
            pl.BlockSpec((1, 8), lambda i: (0, 0)),
        ],
        out_specs=pl.BlockSpec((bm, 8), lambda i: (i, 0)),
        out_shape=jax.ShapeDtypeStruct((_B, 8), jnp.float32),
    )(x, w8, b8)


def kernel(h, cell, W, b):
    Bn, C_h, Kn = h.shape
    tab = jnp.transpose(h, (0, 2, 1)).reshape(Bn * Kn, C_h)
    c0 = cell[:, 0].reshape(8, -1)
    c1 = cell[:, 1].reshape(8, -1)
    kl2, al2 = _compute_indices(c0, c1)
    kl = kl2.reshape(Bn)
    al = al2.reshape(Bn)
    h_read = _sc_call(tab, kl, al)
    w8 = jnp.zeros((C_h, 8), jnp.float32).at[:, :3].set(W.T)
    b8 = jnp.zeros((1, 8), jnp.float32).at[0, :3].set(b)
    res = _matvec(h_read, w8, b8)
    return res[:, :3]


# speedup vs baseline: 11.6461x; 1.3312x over previous
import jax
import jax.numpy as jnp
import numpy as np
from jax import lax
from jax.experimental import pallas as pl
from jax.experimental.pallas import tpu as pltpu
from jax.experimental.pallas import tpu_sc as plsc

_B = 16384
_C = 128
_K = 24
_DELTA_OMEGA = float(np.log(30.0) / 23.0)
_TAU_OFFSET = float(np.log(24.0))
_LN2 = float(np.log(2.0))

_P0 = -2.7941536765361863
_P1 = 5.069756316633883
_P2 = -3.5202188381464623
_P3 = 1.6101775468974928
_P4 = -0.40947558576670895
_P5 = 0.04392862784798757

_NC = 2
_NS = 16
_NW = _NC * _NS
_RPT = _B // _NW
_CHB = 128
_NCHB = _RPT // _CHB
_NG = _RPT // 16


def _log_poly(m):
    r = _P5
    r = r * m + _P4
    r = r * m + _P3
    r = r * m + _P2
    r = r * m + _P1
    return r * m + _P0


def _sc_body(tab, cellT, wf, bf, out,
             cbuf, albuf, ilo, ihi, dlo0, dhi0, dlo1, dhi1, wbuf, bbuf, obuf,
             sem0, sem1, sem2, sem3):
    wid = lax.axis_index("s") * _NC + lax.axis_index("c")
    base = wid * _RPT

    pltpu.sync_copy(cellT.at[:, pl.ds(base, _RPT)], cbuf)
    pltpu.sync_copy(wf, wbuf)
    pltpu.sync_copy(bf, bbuf)
    iota = lax.iota(jnp.int32, 16)

    wv = [[wbuf[pl.ds(o * _C + g * 16, 16)] for g in range(_C // 16)]
          for o in range(3)]
    zero = jnp.zeros((16,), jnp.float32)
    bvec = [jnp.where(iota == 0,
                      plsc.load_gather(bbuf, [jnp.full((16,), o, jnp.int32)]),
                      zero)
            for o in range(3)]

    def build(g, c):
        c0 = cbuf[0, pl.ds(g * 16, 16)]
        c1 = cbuf[1, pl.ds(g * 16, 16)]
        x = jnp.maximum(c0 * c1, 1e-20)
        bits = plsc.bitcast(x, jnp.int32)
        e = lax.shift_right_arithmetic(bits, 23) - 127
        mant = plsc.bitcast(
            jnp.bitwise_or(jnp.bitwise_and(bits, 0x7FFFFF), 0x3F800000),
            jnp.float32)
        lnx = (e.astype(jnp.float32) + _log_poly(mant)) * _LN2
        tau = (-0.5 * lnx - _TAU_OFFSET) * (1.0 / _DELTA_OMEGA)
        tcl = jnp.clip(tau, 0.0, float(_K - 2) + 0.999995)
        kl = tcl.astype(jnp.int32)
        al = jnp.clip(tau - kl.astype(jnp.float32), 0.0, 1.0)
        albuf[pl.ds(g * 16, 16)] = al
        idx = (base + g * 16 + iota) * _K + kl
        ilo[pl.ds(g * 16, 16)] = idx
        ihi[pl.ds(g * 16, 16)] = idx + 1
        return c

    lax.fori_loop(0, _NG, build, 0)

    bufs = [(dlo0, dhi0, sem0, sem1), (dlo1, dhi1, sem2, sem3)]

    def start(ch):
        dl, dh, sl, sh = bufs[ch % 2]
        pltpu.async_copy(tab.at[ilo.at[pl.ds(ch * _CHB, _CHB)]], dl, sl)
        pltpu.async_copy(tab.at[ihi.at[pl.ds(ch * _CHB, _CHB)]], dh, sh)

    def wait(ch):
        dl, dh, sl, sh = bufs[ch % 2]
        pltpu.make_async_copy(tab.at[ilo.at[pl.ds(0, _CHB)]], dl, sl).wait()
        pltpu.make_async_copy(tab.at[ihi.at[pl.ds(0, _CHB)]], dh, sh).wait()

    start(0)
    start(1)

    tau_init = (bvec[0], bvec[1], bvec[2])

    for ch in range(_NCHB):
        wait(ch)
        dl, dh, _, _ = bufs[ch % 2]

        def row(i, ov):
            ov0, ov1, ov2 = ov
            a16 = plsc.load_gather(
                albuf,
                [jnp.full((16,), ch * _CHB, jnp.int32) + i])
            t0 = bvec[0]
            t1 = bvec[1]
            t2 = bvec[2]
            for g in range(_C // 16):
                lo = dl[i, pl.ds(g * 16, 16)]
                hi = dh[i, pl.ds(g * 16, 16)]
                hr = lo + a16 * (hi - lo)
                t0 = t0 + hr * wv[0][g]
                t1 = t1 + hr * wv[1][g]
                t2 = t2 + hr * wv[2][g]
            lane = jnp.bitwise_and(i, 15)
            s0 = jnp.sum(t0)
            s1 = jnp.sum(t1)
            s2 = jnp.sum(t2)
            ov0 = jnp.where(iota == lane, s0, ov0)
            ov1 = jnp.where(iota == lane, s1, ov1)
            ov2 = jnp.where(iota == lane, s2, ov2)
            flush = lane == 15

            @pl.when(flush)
            def _():
                st = ch * _CHB + i - 15
                obuf[0, pl.ds(st, 16)] = ov0
                obuf[1, pl.ds(st, 16)] = ov1
                obuf[2, pl.ds(st, 16)] = ov2

            ov0 = jnp.where(flush, zero, ov0)
            ov1 = jnp.where(flush, zero, ov1)
            ov2 = jnp.where(flush, zero, ov2)
            return (ov0, ov1, ov2)

        lax.fori_loop(0, _CHB, row, tau_init)
        if ch + 2 < _NCHB:
            start(ch + 2)

    pltpu.sync_copy(obuf, out.at[:, pl.ds(base, _RPT)])


def _sc_call(tab, cellT, wf, bf):
    mesh = plsc.VectorSubcoreMesh(core_axis_name="c", subcore_axis_name="s")
    return pl.kernel(
        _sc_body,
        out_type=jax.ShapeDtypeStruct((3, _B), jnp.float32),
        mesh=mesh,
        compiler_params=pltpu.CompilerParams(use_tc_tiling_on_sc=False,
                                             needs_layout_passes=False),
        scratch_types=[
            pltpu.VMEM((2, _RPT), jnp.float32),
            pltpu.VMEM((_RPT,), jnp.float32),
            pltpu.VMEM((_RPT,), jnp.int32),
            pltpu.VMEM((_RPT,), jnp.int32),
            pltpu.VMEM((_CHB, _C), jnp.float32),
            pltpu.VMEM((_CHB, _C), jnp.float32),
            pltpu.VMEM((_CHB, _C), jnp.float32),
            pltpu.VMEM((_CHB, _C), jnp.float32),
            pltpu.VMEM((3 * _C,), jnp.float32),
            pltpu.VMEM((16,), jnp.float32),
            pltpu.VMEM((3, _RPT), jnp.float32),
            pltpu.SemaphoreType.DMA,
            pltpu.SemaphoreType.DMA,
            pltpu.SemaphoreType.DMA,
            pltpu.SemaphoreType.DMA,
        ],
    )(tab, cellT, wf, bf)


def kernel(h, cell, W, b):
    Bn, C_h, Kn = h.shape
    tab = jnp.transpose(h, (0, 2, 1)).reshape(Bn * Kn, C_h)
    cellT = cell.T
    wf = W.reshape(3 * C_h)
    bf = jnp.zeros((16,), jnp.float32).at[:3].set(b)
    o3 = _sc_call(tab, cellT, wf, bf)
    return o3.T


# speedup vs baseline: 13.0077x; 1.1169x over previous
import jax
import jax.numpy as jnp
import numpy as np
from jax import lax
from jax.experimental import pallas as pl
from jax.experimental.pallas import tpu as pltpu
from jax.experimental.pallas import tpu_sc as plsc

_B = 16384
_C = 128
_K = 24
_DELTA_OMEGA = float(np.log(30.0) / 23.0)
_TAU_OFFSET = float(np.log(24.0))
_LN2 = float(np.log(2.0))

_P0 = -2.7941536765361863
_P1 = 5.069756316633883
_P2 = -3.5202188381464623
_P3 = 1.6101775468974928
_P4 = -0.40947558576670895
_P5 = 0.04392862784798757

_NC = 2
_NS = 16
_NW = _NC * _NS
_RPT = _B // _NW
_CHB = 128
_NCHB = _RPT // _CHB
_NG = _RPT // 16


def _log_poly(m):
    r = _P5
    r = r * m + _P4
    r = r * m + _P3
    r = r * m + _P2
    r = r * m + _P1
    return r * m + _P0


def _sc_body(tab, cellT, wf, bf, out,
             cbuf, albuf, ilo, ihi, dlo0, dhi0, dlo1, dhi1, wbuf, bbuf, obuf,
             sem0, sem1, sem2, sem3):
    wid = lax.axis_index("s") * _NC + lax.axis_index("c")
    base = wid * _RPT

    pltpu.sync_copy(cellT.at[:, pl.ds(base, _RPT)], cbuf)
    pltpu.sync_copy(wf, wbuf)
    pltpu.sync_copy(bf, bbuf)
    iota = lax.iota(jnp.int32, 16)

    wv = [[wbuf[pl.ds(o * _C + g * 16, 16)] for g in range(_C // 16)]
          for o in range(3)]
    zero = jnp.zeros((16,), jnp.float32)
    bvec = [jnp.where(iota == 0,
                      plsc.load_gather(bbuf, [jnp.full((16,), o, jnp.int32)]),
                      zero)
            for o in range(3)]

    def build(g, c):
        c0 = cbuf[0, pl.ds(g * 16, 16)]
        c1 = cbuf[1, pl.ds(g * 16, 16)]
        x = jnp.maximum(c0 * c1, 1e-20)
        bits = plsc.bitcast(x, jnp.int32)
        e = lax.shift_right_arithmetic(bits, 23) - 127
        mant = plsc.bitcast(
            jnp.bitwise_or(jnp.bitwise_and(bits, 0x7FFFFF), 0x3F800000),
            jnp.float32)
        lnx = (e.astype(jnp.float32) + _log_poly(mant)) * _LN2
        tau = (-0.5 * lnx - _TAU_OFFSET) * (1.0 / _DELTA_OMEGA)
        tcl = jnp.clip(tau, 0.0, float(_K - 2) + 0.999995)
        kl = tcl.astype(jnp.int32)
        al = jnp.clip(tau - kl.astype(jnp.float32), 0.0, 1.0)
        albuf[pl.ds(g * 16, 16)] = al
        idx = (base + g * 16 + iota) * _K + kl
        ilo[pl.ds(g * 16, 16)] = idx
        ihi[pl.ds(g * 16, 16)] = idx + 1
        return c

    lax.fori_loop(0, _NG, build, 0)

    bufs = [(dlo0, dhi0, sem0, sem1), (dlo1, dhi1, sem2, sem3)]

    def start(ch):
        dl, dh, sl, sh = bufs[ch % 2]
        pltpu.async_copy(tab.at[ilo.at[pl.ds(ch * _CHB, _CHB)]], dl, sl)
        pltpu.async_copy(tab.at[ihi.at[pl.ds(ch * _CHB, _CHB)]], dh, sh)

    def wait(ch):
        dl, dh, sl, sh = bufs[ch % 2]
        pltpu.make_async_copy(tab.at[ilo.at[pl.ds(0, _CHB)]], dl, sl).wait()
        pltpu.make_async_copy(tab.at[ihi.at[pl.ds(0, _CHB)]], dh, sh).wait()

    start(0)
    start(1)

    tau_init = (bvec[0], bvec[1], bvec[2])

    for ch in range(_NCHB):
        wait(ch)
        dl, dh, _, _ = bufs[ch % 2]

        def pair(j, ov):
            ov0, ov1, ov2 = ov
            sums = []
            for u in range(2):
                i = 2 * j + u
                a16 = plsc.load_gather(
                    albuf,
                    [jnp.full((16,), ch * _CHB, jnp.int32) + i])
                t0 = bvec[0]
                t1 = bvec[1]
                t2 = bvec[2]
                for g in range(_C // 16):
                    lo = dl[i, pl.ds(g * 16, 16)]
                    hi = dh[i, pl.ds(g * 16, 16)]
                    hr = lo + a16 * (hi - lo)
                    t0 = t0 + hr * wv[0][g]
                    t1 = t1 + hr * wv[1][g]
                    t2 = t2 + hr * wv[2][g]
                sums.append((jnp.sum(t0), jnp.sum(t1), jnp.sum(t2)))
            laneA = jnp.bitwise_and(2 * j, 15)
            laneB = laneA + 1
            (sa0, sa1, sa2), (sb0, sb1, sb2) = sums
            ov0 = jnp.where(iota == laneA, sa0, ov0)
            ov1 = jnp.where(iota == laneA, sa1, ov1)
            ov2 = jnp.where(iota == laneA, sa2, ov2)
            ov0 = jnp.where(iota == laneB, sb0, ov0)
            ov1 = jnp.where(iota == laneB, sb1, ov1)
            ov2 = jnp.where(iota == laneB, sb2, ov2)
            flush = laneB == 15

            @pl.when(flush)
            def _():
                st = ch * _CHB + 2 * j - 14
                obuf[0, pl.ds(st, 16)] = ov0
                obuf[1, pl.ds(st, 16)] = ov1
                obuf[2, pl.ds(st, 16)] = ov2

            ov0 = jnp.where(flush, zero, ov0)
            ov1 = jnp.where(flush, zero, ov1)
            ov2 = jnp.where(flush, zero, ov2)
            return (ov0, ov1, ov2)

        lax.fori_loop(0, _CHB // 2, pair, tau_init)
        if ch + 2 < _NCHB:
            start(ch + 2)

    pltpu.sync_copy(obuf, out.at[:, pl.ds(base, _RPT)])


def _sc_call(tab, cellT, wf, bf):
    mesh = plsc.VectorSubcoreMesh(core_axis_name="c", subcore_axis_name="s")
    return pl.kernel(
        _sc_body,
        out_type=jax.ShapeDtypeStruct((3, _B), jnp.float32),
        mesh=mesh,
        compiler_params=pltpu.CompilerParams(use_tc_tiling_on_sc=False,
                                             needs_layout_passes=False),
        scratch_types=[
            pltpu.VMEM((2, _RPT), jnp.float32),
            pltpu.VMEM((_RPT,), jnp.float32),
            pltpu.VMEM((_RPT,), jnp.int32),
            pltpu.VMEM((_RPT,), jnp.int32),
            pltpu.VMEM((_CHB, _C), jnp.float32),
            pltpu.VMEM((_CHB, _C), jnp.float32),
            pltpu.VMEM((_CHB, _C), jnp.float32),
            pltpu.VMEM((_CHB, _C), jnp.float32),
            pltpu.VMEM((3 * _C,), jnp.float32),
            pltpu.VMEM((16,), jnp.float32),
            pltpu.VMEM((3, _RPT), jnp.float32),
            pltpu.SemaphoreType.DMA,
            pltpu.SemaphoreType.DMA,
            pltpu.SemaphoreType.DMA,
            pltpu.SemaphoreType.DMA,
        ],
    )(tab, cellT, wf, bf)


def kernel(h, cell, W, b):
    Bn, C_h, Kn = h.shape
    tab = jnp.transpose(h, (0, 2, 1)).reshape(Bn * Kn, C_h)
    cellT = cell.T
    wf = W.reshape(3 * C_h)
    bf = jnp.zeros((16,), jnp.float32).at[:3].set(b)
    o3 = _sc_call(tab, cellT, wf, bf)
    return o3.T


# speedup vs baseline: 14.3301x; 1.1017x over previous
import jax
import jax.numpy as jnp
import numpy as np
from jax import lax
from jax.experimental import pallas as pl
from jax.experimental.pallas import tpu as pltpu
from jax.experimental.pallas import tpu_sc as plsc

_B = 16384
_C = 128
_K = 24
_DELTA_OMEGA = float(np.log(30.0) / 23.0)
_TAU_OFFSET = float(np.log(24.0))
_LN2 = float(np.log(2.0))

_P0 = -2.7941536765361863
_P1 = 5.069756316633883
_P2 = -3.5202188381464623
_P3 = 1.6101775468974928
_P4 = -0.40947558576670895
_P5 = 0.04392862784798757

_NC = 2
_NS = 16
_NW = _NC * _NS
_RPT = _B // _NW
_CHB = 128
_NCHB = _RPT // _CHB
_NG = _RPT // 16


def _log_poly(m):
    r = _P5
    r = r * m + _P4
    r = r * m + _P3
    r = r * m + _P2
    r = r * m + _P1
    return r * m + _P0


def _sc_body(tab, cellT, wf, bf, out,
             cbuf, albuf, ilo, ihi, dlo0, dhi0, dlo1, dhi1, wbuf, bbuf, obuf,
             sem0, sem1, sem2, sem3):
    wid = lax.axis_index("s") * _NC + lax.axis_index("c")
    base = wid * _RPT

    pltpu.sync_copy(cellT.at[:, pl.ds(base, _RPT)], cbuf)
    pltpu.sync_copy(wf, wbuf)
    pltpu.sync_copy(bf, bbuf)
    iota = lax.iota(jnp.int32, 16)

    wv = [[wbuf[pl.ds(o * _C + g * 16, 16)] for g in range(_C // 16)]
          for o in range(3)]
    zero = jnp.zeros((16,), jnp.float32)
    bvec = [jnp.where(iota == 0,
                      plsc.load_gather(bbuf, [jnp.full((16,), o, jnp.int32)]),
                      zero)
            for o in range(3)]

    def build(g, c):
        c0 = cbuf[0, pl.ds(g * 16, 16)]
        c1 = cbuf[1, pl.ds(g * 16, 16)]
        x = jnp.maximum(c0 * c1, 1e-20)
        bits = plsc.bitcast(x, jnp.int32)
        e = lax.shift_right_arithmetic(bits, 23) - 127
        mant = plsc.bitcast(
            jnp.bitwise_or(jnp.bitwise_and(bits, 0x7FFFFF), 0x3F800000),
            jnp.float32)
        lnx = (e.astype(jnp.float32) + _log_poly(mant)) * _LN2
        tau = (-0.5 * lnx - _TAU_OFFSET) * (1.0 / _DELTA_OMEGA)
        tcl = jnp.clip(tau, 0.0, float(_K - 2) + 0.999995)
        kl = tcl.astype(jnp.int32)
        al = jnp.clip(tau - kl.astype(jnp.float32), 0.0, 1.0)
        albuf[pl.ds(g * 16, 16)] = al
        idx = (base + g * 16 + iota) * _K + kl
        ilo[pl.ds(g * 16, 16)] = idx
        ihi[pl.ds(g * 16, 16)] = idx + 1
        return c

    lax.fori_loop(0, _NG, build, 0)

    bufs = [(dlo0, dhi0, sem0, sem1), (dlo1, dhi1, sem2, sem3)]

    def start(ch):
        dl, dh, sl, sh = bufs[ch % 2]
        pltpu.async_copy(tab.at[ilo.at[pl.ds(ch * _CHB, _CHB)]], dl, sl)
        pltpu.async_copy(tab.at[ihi.at[pl.ds(ch * _CHB, _CHB)]], dh, sh)

    def wait(ch):
        dl, dh, sl, sh = bufs[ch % 2]
        pltpu.make_async_copy(tab.at[ilo.at[pl.ds(0, _CHB)]], dl, sl).wait()
        pltpu.make_async_copy(tab.at[ihi.at[pl.ds(0, _CHB)]], dh, sh).wait()

    start(0)
    start(1)

    tau_init = (bvec[0], bvec[1], bvec[2])

    for ch in range(_NCHB):
        wait(ch)
        dl, dh, _, _ = bufs[ch % 2]

        def pair(j, ov):
            ov0, ov1, ov2 = ov
            sums = []
            for u in range(4):
                i = 4 * j + u
                a16 = plsc.load_gather(
                    albuf,
                    [jnp.full((16,), ch * _CHB, jnp.int32) + i])
                t0 = bvec[0]
                t1 = bvec[1]
                t2 = bvec[2]
                for g in range(_C // 16):
                    lo = dl[i, pl.ds(g * 16, 16)]
                    hi = dh[i, pl.ds(g * 16, 16)]
                    hr = lo + a16 * (hi - lo)
                    t0 = t0 + hr * wv[0][g]
                    t1 = t1 + hr * wv[1][g]
                    t2 = t2 + hr * wv[2][g]
                sums.append((jnp.sum(t0), jnp.sum(t1), jnp.sum(t2)))
            lane0 = jnp.bitwise_and(4 * j, 15)
            for u in range(4):
                su0, su1, su2 = sums[u]
                ov0 = jnp.where(iota == lane0 + u, su0, ov0)
                ov1 = jnp.where(iota == lane0 + u, su1, ov1)
                ov2 = jnp.where(iota == lane0 + u, su2, ov2)
            flush = lane0 == 12

            @pl.when(flush)
            def _():
                st = ch * _CHB + 4 * j - 12
                obuf[0, pl.ds(st, 16)] = ov0
                obuf[1, pl.ds(st, 16)] = ov1
                obuf[2, pl.ds(st, 16)] = ov2

            ov0 = jnp.where(flush, zero, ov0)
            ov1 = jnp.where(flush, zero, ov1)
            ov2 = jnp.where(flush, zero, ov2)
            return (ov0, ov1, ov2)

        lax.fori_loop(0, _CHB // 4, pair, tau_init)
        if ch + 2 < _NCHB:
            start(ch + 2)

    pltpu.sync_copy(obuf, out.at[:, pl.ds(base, _RPT)])


def _sc_call(tab, cellT, wf, bf):
    mesh = plsc.VectorSubcoreMesh(core_axis_name="c", subcore_axis_name="s")
    return pl.kernel(
        _sc_body,
        out_type=jax.ShapeDtypeStruct((3, _B), jnp.float32),
        mesh=mesh,
        compiler_params=pltpu.CompilerParams(use_tc_tiling_on_sc=False,
                                             needs_layout_passes=False),
        scratch_types=[
            pltpu.VMEM((2, _RPT), jnp.float32),
            pltpu.VMEM((_RPT,), jnp.float32),
            pltpu.VMEM((_RPT,), jnp.int32),
            pltpu.VMEM((_RPT,), jnp.int32),
            pltpu.VMEM((_CHB, _C), jnp.float32),
            pltpu.VMEM((_CHB, _C), jnp.float32),
            pltpu.VMEM((_CHB, _C), jnp.float32),
            pltpu.VMEM((_CHB, _C), jnp.float32),
            pltpu.VMEM((3 * _C,), jnp.float32),
            pltpu.VMEM((16,), jnp.float32),
            pltpu.VMEM((3, _RPT), jnp.float32),
            pltpu.SemaphoreType.DMA,
            pltpu.SemaphoreType.DMA,
            pltpu.SemaphoreType.DMA,
            pltpu.SemaphoreType.DMA,
        ],
    )(tab, cellT, wf, bf)


def kernel(h, cell, W, b):
    Bn, C_h, Kn = h.shape
    tab = jnp.transpose(h, (0, 2, 1)).reshape(Bn * Kn, C_h)
    cellT = cell.T
    wf = W.reshape(3 * C_h)
    bf = jnp.zeros((16,), jnp.float32).at[:3].set(b)
    o3 = _sc_call(tab, cellT, wf, bf)
    return o3.T
